# Initial kernel scaffold; baseline (speedup 1.0000x reference)
import jax
import jax.numpy as jnp
from jax.experimental import pallas as pl


def kernel(features, edge_index, W1, b1, W2, b2):
    raise NotImplementedError("write your pallas kernel here")


# speedup vs baseline: 5.2788x; 5.2788x over previous
import dataclasses
import functools

import jax
import jax.numpy as jnp
from jax import lax
from jax.experimental import pallas as pl
from jax.experimental.pallas import tpu as pltpu
from jax.experimental.pallas import tpu_sc as plsc

def _sc_compiler_params(tc_tiling=True):
    cp = pltpu.CompilerParams()
    if "needs_layout_passes" in pltpu.CompilerParams.__dataclass_fields__:
        cp = dataclasses.replace(cp, needs_layout_passes=False)
    if not tc_tiling:
        cp = dataclasses.replace(cp, use_tc_tiling_on_sc=False)
    return cp


NC = 2
NS = 16
NW = NC * NS
LANES = 128


def _make_deg_kernel(N, E):
    EPW = E // NW
    HR = (N + LANES - 1) // LANES
    mesh = plsc.VectorSubcoreMesh(core_axis_name="c", subcore_axis_name="s")

    @functools.partial(
        pl.kernel,
        out_type=jax.ShapeDtypeStruct((2 * NW, HR, LANES), jnp.float32),
        mesh=mesh,
        scratch_types=[
            pltpu.VMEM((EPW,), jnp.int32),
            pltpu.VMEM((EPW,), jnp.int32),
            pltpu.VMEM((HR, LANES), jnp.float32),
            pltpu.VMEM((HR, LANES), jnp.float32),
        ],
        compiler_params=_sc_compiler_params(),
    )
    def deg_kernel(src_hbm, dst_hbm, out_hbm, src_v, dst_v, hs_v, hd_v):
        c = lax.axis_index("c")
        s = lax.axis_index("s")
        wid = s * NC + c
        base = wid * EPW
        pltpu.sync_copy(src_hbm.at[pl.ds(base, EPW)], src_v)
        pltpu.sync_copy(dst_hbm.at[pl.ds(base, EPW)], dst_v)

        zeros16 = jnp.zeros((16,), jnp.float32)

        @pl.loop(0, HR)
        def _(i):
            @pl.loop(0, LANES, step=16)
            def _(j):
                hs_v[i, pl.ds(j, 16)] = zeros16
                hd_v[i, pl.ds(j, 16)] = zeros16

        ones16 = jnp.ones((16,), jnp.float32)

        @pl.loop(0, EPW, step=16)
        def _(i):
            sv = src_v[pl.ds(i, 16)]
            dv = dst_v[pl.ds(i, 16)]
            plsc.addupdate_scatter(
                hs_v,
                [lax.shift_right_logical(sv, 7), lax.bitwise_and(sv, 127)],
                ones16,
            )
            plsc.addupdate_scatter(
                hd_v,
                [lax.shift_right_logical(dv, 7), lax.bitwise_and(dv, 127)],
                ones16,
            )

        pltpu.sync_copy(hs_v, out_hbm.at[wid])
        pltpu.sync_copy(hd_v, out_hbm.at[NW + wid])

    return deg_kernel


def _make_agg_kernel(N, E, D, K=80):
    EPW = E // NW
    NCH = EPW // K
    HR = (N + LANES - 1) // LANES
    NPAD = HR * LANES
    NPT = NPAD // NS
    mesh = plsc.VectorSubcoreMesh(core_axis_name="c", subcore_axis_name="s")

    @functools.partial(
        pl.kernel,
        out_type=jax.ShapeDtypeStruct((NC * NPAD, D), jnp.float32),
        mesh=mesh,
        scratch_types=[
            pltpu.VMEM((K,), jnp.int32),
            pltpu.VMEM((K,), jnp.int32),
            pltpu.VMEM((K, D), jnp.float32),
            pltpu.VMEM((8, D), jnp.float32),
            pltpu.VMEM_SHARED((NPAD, D), jnp.float32),
            pltpu.SemaphoreType.DMA,
            pltpu.SemaphoreType.DMA,
        ],
        compiler_params=_sc_compiler_params(tc_tiling=(D % LANES == 0)),
    )
    def agg_kernel(h_hbm, src_hbm, dst_hbm, out_hbm,
                   src_v, dst_v, rows_v, z_v, acc_sp, sem_g, sem_s):
        c = lax.axis_index("c")
        s = lax.axis_index("s")
        wid = s * NC + c

        zeros16 = jnp.zeros((16,), jnp.float32)

        @pl.loop(0, 8)
        def _(i):
            @pl.loop(0, D, step=16)
            def _(j):
                z_v[i, pl.ds(j, 16)] = zeros16

        @pl.loop(0, NPT, step=8)
        def _(r):
            pltpu.sync_copy(z_v, acc_sp.at[pl.ds(s * NPT + r, 8)])

        plsc.subcore_barrier()

        base0 = wid * EPW

        @pl.loop(0, NCH)
        def _(ch):
            b = base0 + ch * K
            pltpu.sync_copy(src_hbm.at[pl.ds(b, K)], src_v)
            pltpu.sync_copy(dst_hbm.at[pl.ds(b, K)], dst_v)
            pltpu.async_copy(h_hbm.at[src_v], rows_v, sem_g).wait()
            pltpu.async_copy(rows_v, acc_sp.at[dst_v], sem_s, add=True).wait()

        plsc.subcore_barrier()
        pltpu.sync_copy(acc_sp.at[pl.ds(s * NPT, NPT)],
                        out_hbm.at[pl.ds(c * NPAD + s * NPT, NPT)])

    return agg_kernel


def _degsum_body(sp_ref, dp_ref, ro_ref, ri_ref):
    so = jnp.sum(sp_ref[...], axis=0)
    si = jnp.sum(dp_ref[...], axis=0)
    ro_ref[...] = lax.rsqrt(jnp.maximum(so, 1.0))
    ri_ref[...] = lax.rsqrt(jnp.maximum(si, 1.0))


def _lin1_body(x_ref, w_ref, ro_ref, h_ref):
    h_ref[...] = jnp.dot(x_ref[...] * ro_ref[...], w_ref[...],
                         preferred_element_type=jnp.float32)


def _mid_body(agg_ref, ri_ref, ro_ref, b1_ref, w2_ref, h2_ref):
    a = agg_ref[0] + agg_ref[1]
    h1 = (a * ri_ref[...] + b1_ref[...]) * ro_ref[...]
    h2_ref[...] = jnp.dot(h1, w2_ref[...], preferred_element_type=jnp.float32)


def _out_body(agg_ref, ri_ref, b2_ref, o_ref):
    a = agg_ref[0] + agg_ref[1]
    o_ref[...] = a * ri_ref[...] + b2_ref[...]


def kernel(features, edge_index, W1, b1, W2, b2):
    N, D_IN = features.shape
    E = edge_index.shape[1]
    D_H = W1.shape[1]
    D_OUT = W2.shape[1]
    HR = (N + LANES - 1) // LANES
    NPAD = HR * LANES

    src = edge_index[0]
    dst = edge_index[1]

    degpart = _make_deg_kernel(N, E)(src, dst)
    degflat = degpart.reshape(2 * NW, NPAD)
    ro_pad, ri_pad = pl.pallas_call(
        _degsum_body,
        out_shape=(jax.ShapeDtypeStruct((NPAD,), jnp.float32),
                   jax.ShapeDtypeStruct((NPAD,), jnp.float32)),
    )(degflat[:NW], degflat[NW:])
    ro = ro_pad.reshape(NPAD, 1)[:N]
    ri = ri_pad.reshape(NPAD, 1)[:N]

    h = pl.pallas_call(
        _lin1_body,
        out_shape=jax.ShapeDtypeStruct((N, D_H), jnp.float32),
    )(features, W1, ro)

    agg1 = _make_agg_kernel(N, E, D_H)(h, src, dst)
    agg1 = agg1.reshape(NC, NPAD, D_H)[:, :N]

    h2 = pl.pallas_call(
        _mid_body,
        out_shape=jax.ShapeDtypeStruct((N, D_OUT), jnp.float32),
    )(agg1, ri, ro, b1.reshape(1, D_H), W2)

    agg2 = _make_agg_kernel(N, E, D_OUT)(h2, src, dst)
    agg2 = agg2.reshape(NC, NPAD, D_OUT)[:, :N]

    out = pl.pallas_call(
        _out_body,
        out_shape=jax.ShapeDtypeStruct((N, D_OUT), jnp.float32),
    )(agg2, ri, b2.reshape(1, D_OUT))

    return out


# speedup vs baseline: 11.1198x; 2.1065x over previous
import dataclasses
import functools

import jax
import jax.numpy as jnp
from jax import lax
from jax.experimental import pallas as pl
from jax.experimental.pallas import tpu as pltpu
from jax.experimental.pallas import tpu_sc as plsc

def _sc_compiler_params(tc_tiling=True):
    cp = pltpu.CompilerParams()
    if "needs_layout_passes" in pltpu.CompilerParams.__dataclass_fields__:
        cp = dataclasses.replace(cp, needs_layout_passes=False)
    if not tc_tiling:
        cp = dataclasses.replace(cp, use_tc_tiling_on_sc=False)
    return cp


NC = 2
NS = 16
NW = NC * NS
LANES = 128


def _make_deg_kernel(N, E):
    EPW = E // NW
    HR = (N + LANES - 1) // LANES
    mesh = plsc.VectorSubcoreMesh(core_axis_name="c", subcore_axis_name="s")

    @functools.partial(
        pl.kernel,
        out_type=jax.ShapeDtypeStruct((2 * NW, HR, LANES), jnp.float32),
        mesh=mesh,
        scratch_types=[
            pltpu.VMEM((EPW,), jnp.int32),
            pltpu.VMEM((EPW,), jnp.int32),
            pltpu.VMEM((HR, LANES), jnp.float32),
            pltpu.VMEM((HR, LANES), jnp.float32),
        ],
        compiler_params=_sc_compiler_params(),
    )
    def deg_kernel(src_hbm, dst_hbm, out_hbm, src_v, dst_v, hs_v, hd_v):
        c = lax.axis_index("c")
        s = lax.axis_index("s")
        wid = s * NC + c
        base = wid * EPW
        pltpu.sync_copy(src_hbm.at[pl.ds(base, EPW)], src_v)
        pltpu.sync_copy(dst_hbm.at[pl.ds(base, EPW)], dst_v)

        zeros16 = jnp.zeros((16,), jnp.float32)

        @pl.loop(0, HR)
        def _(i):
            @pl.loop(0, LANES, step=16)
            def _(j):
                hs_v[i, pl.ds(j, 16)] = zeros16
                hd_v[i, pl.ds(j, 16)] = zeros16

        ones16 = jnp.ones((16,), jnp.float32)

        @pl.loop(0, EPW, step=16)
        def _(i):
            sv = src_v[pl.ds(i, 16)]
            dv = dst_v[pl.ds(i, 16)]
            plsc.addupdate_scatter(
                hs_v,
                [lax.shift_right_logical(sv, 7), lax.bitwise_and(sv, 127)],
                ones16,
            )
            plsc.addupdate_scatter(
                hd_v,
                [lax.shift_right_logical(dv, 7), lax.bitwise_and(dv, 127)],
                ones16,
            )

        pltpu.sync_copy(hs_v, out_hbm.at[wid])
        pltpu.sync_copy(hd_v, out_hbm.at[NW + wid])

    return deg_kernel


def _make_agg_kernel(N, E, D, K=80):
    EPW = E // NW
    NCH = EPW // K
    HR = (N + LANES - 1) // LANES
    NPAD = HR * LANES
    NPT = NPAD // NS
    mesh = plsc.VectorSubcoreMesh(core_axis_name="c", subcore_axis_name="s")

    @functools.partial(
        pl.kernel,
        out_type=jax.ShapeDtypeStruct((NC * NPAD, D), jnp.float32),
        mesh=mesh,
        scratch_types=[
            pltpu.VMEM((EPW,), jnp.int32),
            pltpu.VMEM((NCH, K), jnp.int32),
            pltpu.VMEM((K, D), jnp.float32),
            pltpu.VMEM((K, D), jnp.float32),
            pltpu.VMEM((8, D), jnp.float32),
            pltpu.VMEM_SHARED((NPAD, D), jnp.float32),
            pltpu.SemaphoreType.DMA,
            pltpu.SemaphoreType.DMA,
        ],
        compiler_params=_sc_compiler_params(tc_tiling=(D % LANES == 0)),
    )
    def agg_kernel(h_hbm, src_hbm, dst_hbm, out_hbm,
                   src_v, dst_v, rows_a, rows_b, z_v, acc_sp, sem_a, sem_b):
        c = lax.axis_index("c")
        s = lax.axis_index("s")
        wid = s * NC + c

        pltpu.sync_copy(src_hbm.at[wid], src_v)
        pltpu.sync_copy(dst_hbm.at[wid], dst_v)

        zeros16 = jnp.zeros((16,), jnp.float32)

        @pl.loop(0, 8)
        def _(i):
            @pl.loop(0, D, step=16)
            def _(j):
                z_v[i, pl.ds(j, 16)] = zeros16

        @pl.loop(0, NPT, step=8)
        def _(r):
            pltpu.sync_copy(z_v, acc_sp.at[pl.ds(s * NPT + r, 8)])

        plsc.subcore_barrier()

        def gather(ch, buf, sem):
            return pltpu.async_copy(h_hbm.at[src_v.at[pl.ds(ch * K, K)]],
                                    buf, sem)

        def scatter(ch, buf):
            pltpu.sync_copy(buf, acc_sp.at[dst_v.at[ch]], add=True)

        gather(0, rows_a, sem_a)

        @pl.loop(0, NCH, step=2)
        def _(ch):
            @pl.when(ch + 1 < NCH)
            def _():
                gather(ch + 1, rows_b, sem_b)

            pltpu.make_async_copy(h_hbm.at[src_v.at[pl.ds(ch * K, K)]],
                                  rows_a, sem_a).wait()
            scatter(ch, rows_a)

            @pl.when(ch + 2 < NCH)
            def _():
                gather(ch + 2, rows_a, sem_a)

            @pl.when(ch + 1 < NCH)
            def _():
                pltpu.make_async_copy(h_hbm.at[src_v.at[pl.ds((ch + 1) * K, K)]],
                                      rows_b, sem_b).wait()
                scatter(ch + 1, rows_b)

        plsc.subcore_barrier()
        pltpu.sync_copy(acc_sp.at[pl.ds(s * NPT, NPT)],
                        out_hbm.at[pl.ds(c * NPAD + s * NPT, NPT)])

    return agg_kernel


def _degsum_body(sp_ref, dp_ref, ro_ref, ri_ref):
    so = jnp.sum(sp_ref[...], axis=0)
    si = jnp.sum(dp_ref[...], axis=0)
    ro_ref[...] = lax.rsqrt(jnp.maximum(so, 1.0))
    ri_ref[...] = lax.rsqrt(jnp.maximum(si, 1.0))


def _lin1_body(x_ref, w_ref, ro_ref, h_ref):
    h_ref[...] = jnp.dot(x_ref[...] * ro_ref[...], w_ref[...],
                         preferred_element_type=jnp.float32)


def _mid_body(agg_ref, ri_ref, ro_ref, b1_ref, w2_ref, h2_ref):
    a = agg_ref[0] + agg_ref[1]
    h1 = (a * ri_ref[...] + b1_ref[...]) * ro_ref[...]
    h2_ref[...] = jnp.dot(h1, w2_ref[...], preferred_element_type=jnp.float32)


def _out_body(agg_ref, ri_ref, b2_ref, o_ref):
    a = agg_ref[0] + agg_ref[1]
    o_ref[...] = a * ri_ref[...] + b2_ref[...]


def kernel(features, edge_index, W1, b1, W2, b2):
    N, D_IN = features.shape
    E = edge_index.shape[1]
    D_H = W1.shape[1]
    D_OUT = W2.shape[1]
    HR = (N + LANES - 1) // LANES
    NPAD = HR * LANES

    src = edge_index[0]
    dst = edge_index[1]
    K = 80
    NCH = E // NW // K
    src2 = src.reshape(NW, NCH * K)
    dst3 = dst.reshape(NW, NCH, K)

    degpart = _make_deg_kernel(N, E)(src, dst)
    degflat = degpart.reshape(2 * NW, NPAD)
    ro_pad, ri_pad = pl.pallas_call(
        _degsum_body,
        out_shape=(jax.ShapeDtypeStruct((NPAD,), jnp.float32),
                   jax.ShapeDtypeStruct((NPAD,), jnp.float32)),
    )(degflat[:NW], degflat[NW:])
    ro = ro_pad.reshape(NPAD, 1)[:N]
    ri = ri_pad.reshape(NPAD, 1)[:N]

    h = pl.pallas_call(
        _lin1_body,
        out_shape=jax.ShapeDtypeStruct((N, D_H), jnp.float32),
    )(features, W1, ro)

    agg1 = _make_agg_kernel(N, E, D_H, K=K)(h, src2, dst3)
    agg1 = agg1.reshape(NC, NPAD, D_H)[:, :N]

    h2 = pl.pallas_call(
        _mid_body,
        out_shape=jax.ShapeDtypeStruct((N, D_OUT), jnp.float32),
    )(agg1, ri, ro, b1.reshape(1, D_H), W2)

    agg2 = _make_agg_kernel(N, E, D_OUT, K=K)(h2, src2, dst3)
    agg2 = agg2.reshape(NC, NPAD, D_OUT)[:, :N]

    out = pl.pallas_call(
        _out_body,
        out_shape=jax.ShapeDtypeStruct((N, D_OUT), jnp.float32),
    )(agg2, ri, b2.reshape(1, D_OUT))

    return out


# speedup vs baseline: 11.6431x; 1.0471x over previous
import dataclasses
import functools

import jax
import jax.numpy as jnp
from jax import lax
from jax.experimental import pallas as pl
from jax.experimental.pallas import tpu as pltpu
from jax.experimental.pallas import tpu_sc as plsc

def _sc_compiler_params(tc_tiling=True):
    cp = pltpu.CompilerParams()
    if "needs_layout_passes" in pltpu.CompilerParams.__dataclass_fields__:
        cp = dataclasses.replace(cp, needs_layout_passes=False)
    if not tc_tiling:
        cp = dataclasses.replace(cp, use_tc_tiling_on_sc=False)
    return cp


NC = 2
NS = 16
NW = NC * NS
LANES = 128


def _make_deg_kernel(N, E):
    EPW = E // NW
    HR = (N + LANES - 1) // LANES
    mesh = plsc.VectorSubcoreMesh(core_axis_name="c", subcore_axis_name="s")

    @functools.partial(
        pl.kernel,
        out_type=jax.ShapeDtypeStruct((2 * NW, HR, LANES), jnp.float32),
        mesh=mesh,
        scratch_types=[
            pltpu.VMEM((EPW,), jnp.int32),
            pltpu.VMEM((EPW,), jnp.int32),
            pltpu.VMEM((HR, LANES), jnp.float32),
            pltpu.VMEM((HR, LANES), jnp.float32),
        ],
        compiler_params=_sc_compiler_params(),
    )
    def deg_kernel(src_hbm, dst_hbm, out_hbm, src_v, dst_v, hs_v, hd_v):
        c = lax.axis_index("c")
        s = lax.axis_index("s")
        wid = s * NC + c
        base = wid * EPW
        pltpu.sync_copy(src_hbm.at[pl.ds(base, EPW)], src_v)
        pltpu.sync_copy(dst_hbm.at[pl.ds(base, EPW)], dst_v)

        zeros16 = jnp.zeros((16,), jnp.float32)

        @pl.loop(0, HR)
        def _(i):
            @pl.loop(0, LANES, step=16)
            def _(j):
                hs_v[i, pl.ds(j, 16)] = zeros16
                hd_v[i, pl.ds(j, 16)] = zeros16

        ones16 = jnp.ones((16,), jnp.float32)

        @pl.loop(0, EPW, step=16)
        def _(i):
            sv = src_v[pl.ds(i, 16)]
            dv = dst_v[pl.ds(i, 16)]
            plsc.addupdate_scatter(
                hs_v,
                [lax.shift_right_logical(sv, 7), lax.bitwise_and(sv, 127)],
                ones16,
            )
            plsc.addupdate_scatter(
                hd_v,
                [lax.shift_right_logical(dv, 7), lax.bitwise_and(dv, 127)],
                ones16,
            )

        pltpu.sync_copy(hs_v, out_hbm.at[wid])
        pltpu.sync_copy(hd_v, out_hbm.at[NW + wid])

    return deg_kernel


def _make_agg_kernel(N, E, D, K=80):
    EPW = E // NW
    NCH = EPW // K
    HR = (N + LANES - 1) // LANES
    NPAD = HR * LANES
    NPT = NPAD // NS
    mesh = plsc.VectorSubcoreMesh(core_axis_name="c", subcore_axis_name="s")

    @functools.partial(
        pl.kernel,
        out_type=jax.ShapeDtypeStruct((NC * NPAD, D), jnp.float32),
        mesh=mesh,
        scratch_types=[
            pltpu.VMEM((EPW,), jnp.int32),
            pltpu.VMEM((NCH, K), jnp.int32),
            pltpu.VMEM((K, D), jnp.float32),
            pltpu.VMEM((K, D), jnp.float32),
            pltpu.VMEM((8, D), jnp.float32),
            pltpu.VMEM_SHARED((NPAD, D), jnp.float32),
            pltpu.SemaphoreType.DMA,
            pltpu.SemaphoreType.DMA,
        ],
        compiler_params=_sc_compiler_params(tc_tiling=(D % LANES == 0)),
    )
    def agg_kernel(h_hbm, src_hbm, dst_hbm, out_hbm,
                   src_v, dst_v, rows_a, rows_b, z_v, acc_sp, sem_a, sem_b):
        c = lax.axis_index("c")
        s = lax.axis_index("s")
        wid = s * NC + c

        pltpu.sync_copy(src_hbm.at[wid], src_v)
        pltpu.sync_copy(dst_hbm.at[wid], dst_v)

        zeros16 = jnp.zeros((16,), jnp.float32)

        @pl.loop(0, 8)
        def _(i):
            @pl.loop(0, D, step=16)
            def _(j):
                z_v[i, pl.ds(j, 16)] = zeros16

        @pl.loop(0, NPT, step=8)
        def _(r):
            pltpu.sync_copy(z_v, acc_sp.at[pl.ds(s * NPT + r, 8)])

        plsc.subcore_barrier()

        def gather(ch, buf, sem):
            return pltpu.async_copy(h_hbm.at[src_v.at[pl.ds(ch * K, K)]],
                                    buf, sem)

        def scatter(ch, buf):
            pltpu.sync_copy(buf, acc_sp.at[dst_v.at[ch]], add=True)

        gather(0, rows_a, sem_a)

        @pl.loop(0, NCH, step=2)
        def _(ch):
            @pl.when(ch + 1 < NCH)
            def _():
                gather(ch + 1, rows_b, sem_b)

            pltpu.make_async_copy(h_hbm.at[src_v.at[pl.ds(ch * K, K)]],
                                  rows_a, sem_a).wait()
            scatter(ch, rows_a)

            @pl.when(ch + 2 < NCH)
            def _():
                gather(ch + 2, rows_a, sem_a)

            @pl.when(ch + 1 < NCH)
            def _():
                pltpu.make_async_copy(h_hbm.at[src_v.at[pl.ds((ch + 1) * K, K)]],
                                      rows_b, sem_b).wait()
                scatter(ch + 1, rows_b)

        plsc.subcore_barrier()
        pltpu.sync_copy(acc_sp.at[pl.ds(s * NPT, NPT)],
                        out_hbm.at[pl.ds(c * NPAD + s * NPT, NPT)])

    return agg_kernel


def _degsum_body(sp_ref, dp_ref, ro_ref, ri_ref):
    so = jnp.sum(sp_ref[...], axis=0)
    si = jnp.sum(dp_ref[...], axis=0)
    ro_ref[...] = lax.rsqrt(jnp.maximum(so, 1.0))
    ri_ref[...] = lax.rsqrt(jnp.maximum(si, 1.0))


def _lin1_body(x_ref, w_ref, ro_ref, h_ref):
    h_ref[...] = jnp.dot(x_ref[...] * ro_ref[...], w_ref[...],
                         preferred_element_type=jnp.float32)


def _mid_body(agg_ref, ri_ref, ro_ref, b1_ref, w2_ref, h2_ref):
    n = ri_ref.shape[0]
    a = agg_ref[0] + agg_ref[1]
    h1 = (a[:n] * ri_ref[...] + b1_ref[...]) * ro_ref[...]
    h2_ref[...] = jnp.dot(h1, w2_ref[...], preferred_element_type=jnp.float32)


def _out_body(agg_ref, ri_ref, b2_ref, o_ref):
    n = ri_ref.shape[0]
    a = agg_ref[0] + agg_ref[1]
    o_ref[...] = a[:n] * ri_ref[...] + b2_ref[...]


def kernel(features, edge_index, W1, b1, W2, b2):
    N, D_IN = features.shape
    E = edge_index.shape[1]
    D_H = W1.shape[1]
    D_OUT = W2.shape[1]
    HR = (N + LANES - 1) // LANES
    NPAD = HR * LANES

    src = edge_index[0]
    dst = edge_index[1]
    K = 80
    NCH = E // NW // K
    src2 = src.reshape(NW, NCH * K)
    dst3 = dst.reshape(NW, NCH, K)

    degpart = _make_deg_kernel(N, E)(src, dst)
    degflat = degpart.reshape(2 * NW, NPAD)
    ro_pad, ri_pad = pl.pallas_call(
        _degsum_body,
        out_shape=(jax.ShapeDtypeStruct((NPAD,), jnp.float32),
                   jax.ShapeDtypeStruct((NPAD,), jnp.float32)),
    )(degflat[:NW], degflat[NW:])
    ro = ro_pad.reshape(NPAD, 1)[:N]
    ri = ri_pad.reshape(NPAD, 1)[:N]

    h = pl.pallas_call(
        _lin1_body,
        out_shape=jax.ShapeDtypeStruct((N, D_H), jnp.float32),
    )(features, W1, ro)

    agg1 = _make_agg_kernel(N, E, D_H, K=K)(h, src2, dst3)
    agg1 = agg1.reshape(NC, NPAD, D_H)

    h2 = pl.pallas_call(
        _mid_body,
        out_shape=jax.ShapeDtypeStruct((N, D_OUT), jnp.float32),
    )(agg1, ri, ro, b1.reshape(1, D_H), W2)

    agg2 = _make_agg_kernel(N, E, D_OUT, K=K)(h2, src2, dst3)
    agg2 = agg2.reshape(NC, NPAD, D_OUT)

    out = pl.pallas_call(
        _out_body,
        out_shape=jax.ShapeDtypeStruct((N, D_OUT), jnp.float32),
    )(agg2, ri, b2.reshape(1, D_OUT))

    return out


# speedup vs baseline: 12.4791x; 1.0718x over previous
import dataclasses
import functools

import jax
import jax.numpy as jnp
from jax import lax
from jax.experimental import pallas as pl
from jax.experimental.pallas import tpu as pltpu
from jax.experimental.pallas import tpu_sc as plsc


def _sc_compiler_params(tc_tiling=True):
    cp = pltpu.CompilerParams()
    if "needs_layout_passes" in pltpu.CompilerParams.__dataclass_fields__:
        cp = dataclasses.replace(cp, needs_layout_passes=False)
    if not tc_tiling:
        cp = dataclasses.replace(cp, use_tc_tiling_on_sc=False)
    return cp


NC = 2
NS = 16
NW = NC * NS
LANES = 128


def _make_deg_kernel(N, E):
    EPW = E // NW
    HR = (N + LANES - 1) // LANES
    mesh = plsc.VectorSubcoreMesh(core_axis_name="c", subcore_axis_name="s")

    @functools.partial(
        pl.kernel,
        out_type=jax.ShapeDtypeStruct((2 * NW, HR, LANES), jnp.float32),
        mesh=mesh,
        scratch_types=[
            pltpu.VMEM((EPW,), jnp.int32),
            pltpu.VMEM((EPW,), jnp.int32),
            pltpu.VMEM((HR, LANES), jnp.float32),
            pltpu.VMEM((HR, LANES), jnp.float32),
        ],
        compiler_params=_sc_compiler_params(),
    )
    def deg_kernel(edge_hbm, out_hbm, src_v, dst_v, hs_v, hd_v):
        c = lax.axis_index("c")
        s = lax.axis_index("s")
        wid = s * NC + c
        base = wid * EPW
        pltpu.sync_copy(edge_hbm.at[pl.ds(base, EPW)], src_v)
        pltpu.sync_copy(edge_hbm.at[pl.ds(E + base, EPW)], dst_v)

        zeros16 = jnp.zeros((16,), jnp.float32)

        @pl.loop(0, HR)
        def _(i):
            @pl.loop(0, LANES, step=16)
            def _(j):
                hs_v[i, pl.ds(j, 16)] = zeros16
                hd_v[i, pl.ds(j, 16)] = zeros16

        ones16 = jnp.ones((16,), jnp.float32)

        @pl.loop(0, EPW, step=16)
        def _(i):
            sv = src_v[pl.ds(i, 16)]
            dv = dst_v[pl.ds(i, 16)]
            plsc.addupdate_scatter(
                hs_v,
                [lax.shift_right_logical(sv, 7), lax.bitwise_and(sv, 127)],
                ones16,
            )
            plsc.addupdate_scatter(
                hd_v,
                [lax.shift_right_logical(dv, 7), lax.bitwise_and(dv, 127)],
                ones16,
            )

        pltpu.sync_copy(hs_v, out_hbm.at[wid])
        pltpu.sync_copy(hd_v, out_hbm.at[NW + wid])

    return deg_kernel


def _make_agg_kernel(N, E, D, K=80):
    EPW = E // NW
    NCH = EPW // K
    HR = (N + LANES - 1) // LANES
    NPAD = HR * LANES
    NPT = NPAD // NS
    RD = 4
    NB = 2
    mesh = plsc.VectorSubcoreMesh(core_axis_name="c", subcore_axis_name="s")

    @functools.partial(
        pl.kernel,
        out_type=jax.ShapeDtypeStruct((NC * NPAD, D), jnp.float32),
        mesh=mesh,
        scratch_types=[
            pltpu.VMEM((RD, K), jnp.int32),
            pltpu.VMEM((RD, K), jnp.int32),
            pltpu.VMEM((K, D), jnp.float32),
            pltpu.VMEM((K, D), jnp.float32),
            pltpu.VMEM((8, D), jnp.float32),
            pltpu.VMEM_SHARED((NPAD, D), jnp.float32),
            [pltpu.SemaphoreType.DMA] * RD,
            [pltpu.SemaphoreType.DMA] * NB,
        ],
        compiler_params=_sc_compiler_params(tc_tiling=(D % LANES == 0)),
    )
    def agg_kernel(h_hbm, edge_hbm, out_hbm,
                   sring, dring, rows_a, rows_b, z_v, acc_sp, isem, gsem):
        c = lax.axis_index("c")
        s = lax.axis_index("s")
        wid = s * NC + c
        base = wid * EPW
        rows = (rows_a, rows_b)

        def idx_issue(cc, j):
            pltpu.async_copy(edge_hbm.at[pl.ds(base + cc * K, K)],
                             sring.at[j], isem[j])
            pltpu.async_copy(edge_hbm.at[pl.ds(E + base + cc * K, K)],
                             dring.at[j], isem[j])

        def idx_wait(j):
            pltpu.make_async_copy(edge_hbm.at[pl.ds(base, K)],
                                  sring.at[j], isem[j]).wait()
            pltpu.make_async_copy(edge_hbm.at[pl.ds(E + base, K)],
                                  dring.at[j], isem[j]).wait()

        def gather_issue(j, b):
            pltpu.async_copy(h_hbm.at[sring.at[j]], rows[b], gsem[b])

        def gather_wait(j, b):
            pltpu.make_async_copy(h_hbm.at[sring.at[j]], rows[b],
                                  gsem[b]).wait()

        for j in range(RD):
            idx_issue(j, j)

        zeros16 = jnp.zeros((16,), jnp.float32)

        @pl.loop(0, 8)
        def _(i):
            @pl.loop(0, D, step=16)
            def _(j):
                z_v[i, pl.ds(j, 16)] = zeros16

        @pl.loop(0, NPT, step=8)
        def _(r):
            pltpu.sync_copy(z_v, acc_sp.at[pl.ds(s * NPT + r, 8)])

        for b in range(NB):
            idx_wait(b)
            gather_issue(b, b)

        plsc.subcore_barrier()

        @pl.loop(0, NCH, step=RD)
        def _(ch):
            for j in range(RD):
                cc = ch + j
                b = j & 1

                @pl.when(cc < NCH)
                def _():
                    gather_wait(j, b)
                    pltpu.sync_copy(rows[b], acc_sp.at[dring.at[j]], add=True)

                    @pl.when(cc + RD < NCH)
                    def _():
                        idx_issue(cc + RD, j)

                    @pl.when(cc + NB < NCH)
                    def _():
                        idx_wait((j + NB) % RD)
                        gather_issue((j + NB) % RD, b)

        plsc.subcore_barrier()
        pltpu.sync_copy(acc_sp.at[pl.ds(s * NPT, NPT)],
                        out_hbm.at[pl.ds(c * NPAD + s * NPT, NPT)])

    return agg_kernel


def _degsum_body(dp_ref, ro_ref, ri_ref):
    so = jnp.sum(dp_ref[:NW], axis=0)
    si = jnp.sum(dp_ref[NW:], axis=0)
    ro_ref[...] = lax.rsqrt(jnp.maximum(so, 1.0))
    ri_ref[...] = lax.rsqrt(jnp.maximum(si, 1.0))


def _lin1_body(x_ref, w_ref, ro_ref, h_ref):
    n = x_ref.shape[0]
    h_ref[...] = jnp.dot(x_ref[...] * ro_ref[:n], w_ref[...],
                         preferred_element_type=jnp.float32)


def _make_mid_body(NPAD):
    def mid_body(agg_ref, ri_ref, ro_ref, b1_ref, w2_ref, h2_ref):
        n = h2_ref.shape[0]
        a = agg_ref[:n] + agg_ref[pl.ds(NPAD, n)]
        h1 = (a * ri_ref[:n] + b1_ref[...][None, :]) * ro_ref[:n]
        h2_ref[...] = jnp.dot(h1, w2_ref[...],
                              preferred_element_type=jnp.float32)
    return mid_body


def _make_out_body(NPAD):
    def out_body(agg_ref, ri_ref, b2_ref, o_ref):
        n = o_ref.shape[0]
        a = agg_ref[:n] + agg_ref[pl.ds(NPAD, n)]
        o_ref[...] = a * ri_ref[:n] + b2_ref[...][None, :]
    return out_body


def kernel(features, edge_index, W1, b1, W2, b2):
    N, D_IN = features.shape
    E = edge_index.shape[1]
    D_H = W1.shape[1]
    D_OUT = W2.shape[1]
    HR = (N + LANES - 1) // LANES
    NPAD = HR * LANES

    edge_flat = edge_index.reshape(2 * E)

    degpart = _make_deg_kernel(N, E)(edge_flat)
    ro, ri = pl.pallas_call(
        _degsum_body,
        out_shape=(jax.ShapeDtypeStruct((HR, LANES), jnp.float32),
                   jax.ShapeDtypeStruct((HR, LANES), jnp.float32)),
    )(degpart)
    ro = ro.reshape(NPAD, 1)
    ri = ri.reshape(NPAD, 1)

    h = pl.pallas_call(
        _lin1_body,
        out_shape=jax.ShapeDtypeStruct((N, D_H), jnp.float32),
    )(features, W1, ro)

    agg1 = _make_agg_kernel(N, E, D_H)(h, edge_flat)

    h2 = pl.pallas_call(
        _make_mid_body(NPAD),
        out_shape=jax.ShapeDtypeStruct((N, D_OUT), jnp.float32),
    )(agg1, ri, ro, b1, W2)

    agg2 = _make_agg_kernel(N, E, D_OUT)(h2, edge_flat)

    out = pl.pallas_call(
        _make_out_body(NPAD),
        out_shape=jax.ShapeDtypeStruct((N, D_OUT), jnp.float32),
    )(agg2, ri, b2)

    return out


# speedup vs baseline: 15.0965x; 1.2097x over previous
import dataclasses
import functools

import jax
import jax.numpy as jnp
from jax import lax
from jax.experimental import pallas as pl
from jax.experimental.pallas import tpu as pltpu
from jax.experimental.pallas import tpu_sc as plsc


def _sc_compiler_params(tc_tiling=True):
    cp = pltpu.CompilerParams()
    if "needs_layout_passes" in pltpu.CompilerParams.__dataclass_fields__:
        cp = dataclasses.replace(cp, needs_layout_passes=False)
    if not tc_tiling:
        cp = dataclasses.replace(cp, use_tc_tiling_on_sc=False)
    return cp


NC = 2
NS = 16
NW = NC * NS
LANES = 128


def _make_deg_kernel(N, E):
    EPW = E // NW
    HR = (N + LANES - 1) // LANES
    mesh = plsc.VectorSubcoreMesh(core_axis_name="c", subcore_axis_name="s")

    @functools.partial(
        pl.kernel,
        out_type=jax.ShapeDtypeStruct((2 * NW, HR, LANES), jnp.float32),
        mesh=mesh,
        scratch_types=[
            pltpu.VMEM((EPW,), jnp.int32),
            pltpu.VMEM((EPW,), jnp.int32),
            pltpu.VMEM((HR, LANES), jnp.float32),
            pltpu.VMEM((HR, LANES), jnp.float32),
        ],
        compiler_params=_sc_compiler_params(),
    )
    def deg_kernel(edge_hbm, out_hbm, src_v, dst_v, hs_v, hd_v):
        c = lax.axis_index("c")
        s = lax.axis_index("s")
        wid = s * NC + c
        base = wid * EPW
        pltpu.sync_copy(edge_hbm.at[pl.ds(base, EPW)], src_v)
        pltpu.sync_copy(edge_hbm.at[pl.ds(E + base, EPW)], dst_v)

        zeros16 = jnp.zeros((16,), jnp.float32)

        @pl.loop(0, HR)
        def _(i):
            @pl.loop(0, LANES, step=16)
            def _(j):
                hs_v[i, pl.ds(j, 16)] = zeros16
                hd_v[i, pl.ds(j, 16)] = zeros16

        ones16 = jnp.ones((16,), jnp.float32)

        @pl.loop(0, EPW, step=16)
        def _(i):
            sv = src_v[pl.ds(i, 16)]
            dv = dst_v[pl.ds(i, 16)]
            plsc.addupdate_scatter(
                hs_v,
                [lax.shift_right_logical(sv, 7), lax.bitwise_and(sv, 127)],
                ones16,
            )
            plsc.addupdate_scatter(
                hd_v,
                [lax.shift_right_logical(dv, 7), lax.bitwise_and(dv, 127)],
                ones16,
            )

        pltpu.sync_copy(hs_v, out_hbm.at[wid])
        pltpu.sync_copy(hd_v, out_hbm.at[NW + wid])

    return deg_kernel


def _make_agg_kernel(N, E, D, K=80, NB=2):
    EPW = E // NW
    NCH = EPW // K
    HR = (N + LANES - 1) // LANES
    NPAD = HR * LANES
    NPT = NPAD // NS
    RD = 2 * NB
    mesh = plsc.VectorSubcoreMesh(core_axis_name="c", subcore_axis_name="s")

    @functools.partial(
        pl.kernel,
        out_type=jax.ShapeDtypeStruct((NC * NPAD, D), jnp.float32),
        mesh=mesh,
        scratch_types=[
            pltpu.VMEM((RD, K), jnp.int32),
            pltpu.VMEM((RD, K), jnp.int32),
            [pltpu.VMEM((K, D), jnp.float32)] * NB,
            pltpu.VMEM((8, D), jnp.float32),
            pltpu.VMEM_SHARED((NPAD, D), jnp.float32),
            [pltpu.SemaphoreType.DMA] * RD,
            [pltpu.SemaphoreType.DMA] * NB,
        ],
        compiler_params=_sc_compiler_params(tc_tiling=(D % LANES == 0)),
    )
    def agg_kernel(h_hbm, edge_hbm, out_hbm,
                   sring, dring, rows, z_v, acc_sp, isem, gsem):
        c = lax.axis_index("c")
        s = lax.axis_index("s")
        wid = s * NC + c
        base = wid * EPW

        def idx_issue(cc, j):
            pltpu.async_copy(edge_hbm.at[pl.ds(base + cc * K, K)],
                             sring.at[j], isem[j])
            pltpu.async_copy(edge_hbm.at[pl.ds(E + base + cc * K, K)],
                             dring.at[j], isem[j])

        def idx_wait(j):
            pltpu.make_async_copy(edge_hbm.at[pl.ds(base, K)],
                                  sring.at[j], isem[j]).wait()
            pltpu.make_async_copy(edge_hbm.at[pl.ds(E + base, K)],
                                  dring.at[j], isem[j]).wait()

        def gather_issue(j, b):
            pltpu.async_copy(h_hbm.at[sring.at[j]], rows[b], gsem[b])

        def gather_wait(j, b):
            pltpu.make_async_copy(h_hbm.at[sring.at[j]], rows[b],
                                  gsem[b]).wait()

        for j in range(RD):
            idx_issue(j, j)

        zeros16 = jnp.zeros((16,), jnp.float32)

        @pl.loop(0, 8)
        def _(i):
            @pl.loop(0, D, step=16)
            def _(j):
                z_v[i, pl.ds(j, 16)] = zeros16

        @pl.loop(0, NPT, step=8)
        def _(r):
            pltpu.sync_copy(z_v, acc_sp.at[pl.ds(s * NPT + r, 8)])

        for b in range(NB):
            idx_wait(b)
            gather_issue(b, b)

        plsc.subcore_barrier()

        @pl.loop(0, NCH, step=RD)
        def _(ch):
            for j in range(RD):
                cc = ch + j
                b = j % NB

                @pl.when(cc < NCH)
                def _():
                    gather_wait(j, b)
                    pltpu.sync_copy(rows[b], acc_sp.at[dring.at[j]], add=True)

                    @pl.when(cc + RD < NCH)
                    def _():
                        idx_issue(cc + RD, j)

                    @pl.when(cc + NB < NCH)
                    def _():
                        idx_wait((j + NB) % RD)
                        gather_issue((j + NB) % RD, b)

        plsc.subcore_barrier()
        pltpu.sync_copy(acc_sp.at[pl.ds(s * NPT, NPT)],
                        out_hbm.at[pl.ds(c * NPAD + s * NPT, NPT)])

    return agg_kernel


def _make_degsum_body(HR):
    def degsum_body(dp_ref, ro_ref, ri_ref):
        so = jnp.sum(dp_ref[:NW], axis=0)
        si = jnp.sum(dp_ref[NW:], axis=0)
        ro = lax.rsqrt(jnp.maximum(so, 1.0))
        ri = lax.rsqrt(jnp.maximum(si, 1.0))
        for hi in range(HR):
            ro_ref[pl.ds(hi * LANES, LANES), :] = jnp.transpose(
                ro[hi:hi + 1, :])
            ri_ref[pl.ds(hi * LANES, LANES), :] = jnp.transpose(
                ri[hi:hi + 1, :])
    return degsum_body


def _lin1_body(x_ref, w_ref, ro_ref, h_ref):
    n = x_ref.shape[0]
    h_ref[...] = jnp.dot(x_ref[...] * ro_ref[:n], w_ref[...],
                         preferred_element_type=jnp.float32)


def _make_mid_body(NPAD):
    def mid_body(agg_ref, ri_ref, ro_ref, b1_ref, w2_ref, h2_ref):
        n = h2_ref.shape[0]
        a = agg_ref[:n] + agg_ref[pl.ds(NPAD, n)]
        h1 = (a * ri_ref[:n] + b1_ref[...][None, :]) * ro_ref[:n]
        h2_ref[...] = jnp.dot(h1, w2_ref[...],
                              preferred_element_type=jnp.float32)
    return mid_body


def _make_out_body(NPAD):
    def out_body(agg_ref, ri_ref, b2_ref, o_ref):
        n = o_ref.shape[0]
        a = agg_ref[:n] + agg_ref[pl.ds(NPAD, n)]
        o_ref[...] = a * ri_ref[:n] + b2_ref[...][None, :]
    return out_body


def kernel(features, edge_index, W1, b1, W2, b2):
    N, D_IN = features.shape
    E = edge_index.shape[1]
    D_H = W1.shape[1]
    D_OUT = W2.shape[1]
    HR = (N + LANES - 1) // LANES
    NPAD = HR * LANES

    edge_flat = edge_index.reshape(2 * E)

    degpart = _make_deg_kernel(N, E)(edge_flat)
    ro, ri = pl.pallas_call(
        _make_degsum_body(HR),
        out_shape=(jax.ShapeDtypeStruct((NPAD, 1), jnp.float32),
                   jax.ShapeDtypeStruct((NPAD, 1), jnp.float32)),
    )(degpart)

    h = pl.pallas_call(
        _lin1_body,
        out_shape=jax.ShapeDtypeStruct((N, D_H), jnp.float32),
    )(features, W1, ro)

    agg1 = _make_agg_kernel(N, E, D_H, NB=3)(h, edge_flat)

    h2 = pl.pallas_call(
        _make_mid_body(NPAD),
        out_shape=jax.ShapeDtypeStruct((N, D_OUT), jnp.float32),
    )(agg1, ri, ro, b1, W2)

    agg2 = _make_agg_kernel(N, E, D_OUT, NB=4)(h2, edge_flat)

    out = pl.pallas_call(
        _make_out_body(NPAD),
        out_shape=jax.ShapeDtypeStruct((N, D_OUT), jnp.float32),
    )(agg2, ri, b2)

    return out


# speedup vs baseline: 15.4221x; 1.0216x over previous
import dataclasses
import functools

import jax
import jax.numpy as jnp
from jax import lax
from jax.experimental import pallas as pl
from jax.experimental.pallas import tpu as pltpu
from jax.experimental.pallas import tpu_sc as plsc


def _sc_compiler_params(tc_tiling=True):
    cp = pltpu.CompilerParams()
    if "needs_layout_passes" in pltpu.CompilerParams.__dataclass_fields__:
        cp = dataclasses.replace(cp, needs_layout_passes=False)
    if not tc_tiling:
        cp = dataclasses.replace(cp, use_tc_tiling_on_sc=False)
    return cp


NC = 2
NS = 16
NW = NC * NS
LANES = 128


def _make_deg_kernel(N, E):
    EPW = E // NW
    HR = (N + LANES - 1) // LANES
    mesh = plsc.VectorSubcoreMesh(core_axis_name="c", subcore_axis_name="s")

    @functools.partial(
        pl.kernel,
        out_type=jax.ShapeDtypeStruct((2 * NW, HR, LANES), jnp.float32),
        mesh=mesh,
        scratch_types=[
            pltpu.VMEM((EPW,), jnp.int32),
            pltpu.VMEM((EPW,), jnp.int32),
            pltpu.VMEM((HR, LANES), jnp.float32),
            pltpu.VMEM((HR, LANES), jnp.float32),
        ],
        compiler_params=_sc_compiler_params(),
    )
    def deg_kernel(edge_hbm, out_hbm, src_v, dst_v, hs_v, hd_v):
        c = lax.axis_index("c")
        s = lax.axis_index("s")
        wid = s * NC + c
        base = wid * EPW
        pltpu.sync_copy(edge_hbm.at[pl.ds(base, EPW)], src_v)
        pltpu.sync_copy(edge_hbm.at[pl.ds(E + base, EPW)], dst_v)

        zeros16 = jnp.zeros((16,), jnp.float32)

        @pl.loop(0, HR)
        def _(i):
            @pl.loop(0, LANES, step=16)
            def _(j):
                hs_v[i, pl.ds(j, 16)] = zeros16
                hd_v[i, pl.ds(j, 16)] = zeros16

        ones16 = jnp.ones((16,), jnp.float32)

        @pl.loop(0, EPW, step=16)
        def _(i):
            sv = src_v[pl.ds(i, 16)]
            dv = dst_v[pl.ds(i, 16)]
            plsc.addupdate_scatter(
                hs_v,
                [lax.shift_right_logical(sv, 7), lax.bitwise_and(sv, 127)],
                ones16,
            )
            plsc.addupdate_scatter(
                hd_v,
                [lax.shift_right_logical(dv, 7), lax.bitwise_and(dv, 127)],
                ones16,
            )

        pltpu.sync_copy(hs_v, out_hbm.at[wid])
        pltpu.sync_copy(hd_v, out_hbm.at[NW + wid])

    return deg_kernel


def _make_agg_kernel(N, E, D, K=80, NB=2):
    EPW = E // NW
    NCH = EPW // K
    HR = (N + LANES - 1) // LANES
    NPAD = HR * LANES
    NPT = NPAD // NS
    RD = 2 * NB
    mesh = plsc.VectorSubcoreMesh(core_axis_name="c", subcore_axis_name="s")

    @functools.partial(
        pl.kernel,
        out_type=jax.ShapeDtypeStruct((NC * NPAD, D), jnp.float32),
        mesh=mesh,
        scratch_types=[
            pltpu.VMEM((RD, K), jnp.int32),
            pltpu.VMEM((RD, K), jnp.int32),
            [pltpu.VMEM((K, D), jnp.float32)] * NB,
            pltpu.VMEM((8, D), jnp.float32),
            pltpu.VMEM_SHARED((NPAD, D), jnp.float32),
            [pltpu.SemaphoreType.DMA] * RD,
            [pltpu.SemaphoreType.DMA] * NB,
        ],
        compiler_params=_sc_compiler_params(tc_tiling=(D % LANES == 0)),
    )
    def agg_kernel(h_hbm, edge_hbm, out_hbm,
                   sring, dring, rows, z_v, acc_sp, isem, gsem):
        c = lax.axis_index("c")
        s = lax.axis_index("s")
        wid = s * NC + c
        base = wid * EPW

        def idx_issue(cc, j):
            pltpu.async_copy(edge_hbm.at[pl.ds(base + cc * K, K)],
                             sring.at[j], isem[j])
            pltpu.async_copy(edge_hbm.at[pl.ds(E + base + cc * K, K)],
                             dring.at[j], isem[j])

        def idx_wait(j):
            pltpu.make_async_copy(edge_hbm.at[pl.ds(base, K)],
                                  sring.at[j], isem[j]).wait()
            pltpu.make_async_copy(edge_hbm.at[pl.ds(E + base, K)],
                                  dring.at[j], isem[j]).wait()

        def gather_issue(j, b):
            pltpu.async_copy(h_hbm.at[sring.at[j]], rows[b], gsem[b])

        def gather_wait(j, b):
            pltpu.make_async_copy(h_hbm.at[sring.at[j]], rows[b],
                                  gsem[b]).wait()

        for j in range(RD):
            idx_issue(j, j)

        zeros16 = jnp.zeros((16,), jnp.float32)

        @pl.loop(0, 8)
        def _(i):
            @pl.loop(0, D, step=16)
            def _(j):
                z_v[i, pl.ds(j, 16)] = zeros16

        @pl.loop(0, NPT, step=8)
        def _(r):
            pltpu.sync_copy(z_v, acc_sp.at[pl.ds(s * NPT + r, 8)])

        for b in range(NB):
            idx_wait(b)
            gather_issue(b, b)

        plsc.subcore_barrier()

        @pl.loop(0, NCH, step=RD)
        def _(ch):
            for j in range(RD):
                cc = ch + j
                b = j % NB

                @pl.when(cc < NCH)
                def _():
                    gather_wait(j, b)
                    pltpu.sync_copy(rows[b], acc_sp.at[dring.at[j]], add=True)

                    @pl.when(cc + RD < NCH)
                    def _():
                        idx_issue(cc + RD, j)

                    @pl.when(cc + NB < NCH)
                    def _():
                        idx_wait((j + NB) % RD)
                        gather_issue((j + NB) % RD, b)

        plsc.subcore_barrier()
        pltpu.sync_copy(acc_sp.at[pl.ds(s * NPT, NPT)],
                        out_hbm.at[pl.ds(c * NPAD + s * NPT, NPT)])

    return agg_kernel


def _make_degsum_body(HR):
    def degsum_body(dp_ref, ro_ref, ri_ref):
        so = jnp.sum(dp_ref[:NW], axis=0)
        si = jnp.sum(dp_ref[NW:], axis=0)
        ro = lax.rsqrt(jnp.maximum(so, 1.0))
        ri = lax.rsqrt(jnp.maximum(si, 1.0))
        for hi in range(HR):
            ro_ref[pl.ds(hi * LANES, LANES), :] = jnp.transpose(
                ro[hi:hi + 1, :])
            ri_ref[pl.ds(hi * LANES, LANES), :] = jnp.transpose(
                ri[hi:hi + 1, :])
    return degsum_body


def _lin1_body(x_ref, w_ref, ro_ref, h_ref):
    n = x_ref.shape[0]
    h_ref[...] = jnp.dot(x_ref[...] * ro_ref[:n], w_ref[...],
                         preferred_element_type=jnp.float32)


def _make_mid_body(NPAD):
    def mid_body(agg_ref, ri_ref, ro_ref, b1_ref, w2_ref, h2_ref):
        n = h2_ref.shape[0]
        a = agg_ref[:n] + agg_ref[pl.ds(NPAD, n)]
        h1 = (a * ri_ref[:n] + b1_ref[...][None, :]) * ro_ref[:n]
        h2_ref[...] = jnp.dot(h1, w2_ref[...],
                              preferred_element_type=jnp.float32)
    return mid_body


def _make_out_body(NPAD):
    def out_body(agg_ref, ri_ref, b2_ref, o_ref):
        n = o_ref.shape[0]
        a = agg_ref[:n] + agg_ref[pl.ds(NPAD, n)]
        o_ref[...] = a * ri_ref[:n] + b2_ref[...][None, :]
    return out_body


def kernel(features, edge_index, W1, b1, W2, b2):
    N, D_IN = features.shape
    E = edge_index.shape[1]
    D_H = W1.shape[1]
    D_OUT = W2.shape[1]
    HR = (N + LANES - 1) // LANES
    NPAD = HR * LANES

    edge_flat = edge_index.reshape(2 * E)

    degpart = _make_deg_kernel(N, E)(edge_flat)
    ro, ri = pl.pallas_call(
        _make_degsum_body(HR),
        out_shape=(jax.ShapeDtypeStruct((NPAD, 1), jnp.float32),
                   jax.ShapeDtypeStruct((NPAD, 1), jnp.float32)),
    )(degpart)

    h = pl.pallas_call(
        _lin1_body,
        out_shape=jax.ShapeDtypeStruct((N, D_H), jnp.float32),
    )(features, W1, ro)

    agg1 = _make_agg_kernel(N, E, D_H, NB=4)(h, edge_flat)

    h2 = pl.pallas_call(
        _make_mid_body(NPAD),
        out_shape=jax.ShapeDtypeStruct((N, D_OUT), jnp.float32),
    )(agg1, ri, ro, b1, W2)

    agg2 = _make_agg_kernel(N, E, D_OUT, NB=6)(h2, edge_flat)

    out = pl.pallas_call(
        _make_out_body(NPAD),
        out_shape=jax.ShapeDtypeStruct((N, D_OUT), jnp.float32),
    )(agg2, ri, b2)

    return out
